# baseline (speedup 1.0000x reference)
import functools

import jax
import jax.numpy as jnp
from jax import lax
from jax.experimental import pallas as pl
from jax.experimental.pallas import tpu as pltpu
from jax.experimental.pallas import tpu_sc as plsc

H = 128
N = 10000
E = 320000
G = 64
AF, AV = 9, 64
BF, BV = 3, 8
T = BV ** BF

GC = 128
NBLK = E // GC
NC, NS = 2, 16
ROWS_PER_SUB = 624

_HIGH = jax.lax.Precision.HIGHEST

_mesh = plsc.VectorSubcoreMesh(core_axis_name="c", subcore_axis_name="s")


def _sc_gather1(tab, idx):
    @functools.partial(
        pl.kernel,
        out_type=jax.ShapeDtypeStruct((E, H), tab.dtype),
        mesh=_mesh,
    )
    def g1(t_hbm, i_hbm, o_hbm):
        def body(i_v, o_v):
            pltpu.sync_copy(t_hbm.at[i_v.at[0]], o_v)

        pltpu.emit_pipeline(
            body,
            grid=(NBLK,),
            in_specs=[pl.BlockSpec((1, GC), lambda i: (i, 0))],
            out_specs=[pl.BlockSpec((GC, H), lambda i: (i, 0))],
            core_axis_name=("c", "s"),
            dimension_semantics=(pltpu.PARALLEL,),
        )(i_hbm, o_hbm)

    return g1(tab, idx)


def _sc_gather2(tab_a, idx_a, tab_b, idx_b):
    @functools.partial(
        pl.kernel,
        out_type=(jax.ShapeDtypeStruct((E, H), tab_a.dtype),
                  jax.ShapeDtypeStruct((E, H), tab_b.dtype)),
        mesh=_mesh,
    )
    def g2(ta_hbm, ia_hbm, tb_hbm, ib_hbm, oa_hbm, ob_hbm):
        def body(ia_v, ib_v, oa_v, ob_v):
            pltpu.sync_copy(ta_hbm.at[ia_v.at[0]], oa_v)
            pltpu.sync_copy(tb_hbm.at[ib_v.at[0]], ob_v)

        pltpu.emit_pipeline(
            body,
            grid=(NBLK,),
            in_specs=[
                pl.BlockSpec((1, GC), lambda i: (i, 0)),
                pl.BlockSpec((1, GC), lambda i: (i, 0)),
            ],
            out_specs=[
                pl.BlockSpec((GC, H), lambda i: (i, 0)),
                pl.BlockSpec((GC, H), lambda i: (i, 0)),
            ],
            core_axis_name=("c", "s"),
            dimension_semantics=(pltpu.PARALLEL,),
        )(ia_hbm, ib_hbm, oa_hbm, ob_hbm)

    return g2(tab_a, idx_a, tab_b, idx_b)


def _sc_scatter_add(vals, idx, zeros):
    @functools.partial(
        pl.kernel,
        out_type=jax.ShapeDtypeStruct((NC, N, H), jnp.float32),
        mesh=_mesh,
        scratch_types=[pltpu.VMEM_SHARED((N, H), jnp.float32)],
    )
    def sk(v_hbm, i_hbm, z_hbm, o_hbm, acc):
        cid = lax.axis_index("c")
        sid = lax.axis_index("s")
        sl = pl.ds(sid * ROWS_PER_SUB, ROWS_PER_SUB)
        sl_last = pl.ds((NS - 1) * ROWS_PER_SUB, N - (NS - 1) * ROWS_PER_SUB)

        @pl.when(sid < NS - 1)
        def _():
            pltpu.sync_copy(z_hbm.at[sl], acc.at[sl])

        @pl.when(sid == NS - 1)
        def _():
            pltpu.sync_copy(z_hbm.at[sl_last], acc.at[sl_last])

        plsc.subcore_barrier()

        def body(v_v, i_v):
            pltpu.sync_copy(v_v, acc.at[i_v.at[0]], add=True)

        pltpu.emit_pipeline(
            body,
            grid=(NBLK,),
            in_specs=[
                pl.BlockSpec((GC, H), lambda i: (i, 0)),
                pl.BlockSpec((1, GC), lambda i: (i, 0)),
            ],
            out_specs=[],
            core_axis_name=("c", "s"),
            dimension_semantics=(pltpu.PARALLEL,),
        )(v_hbm, i_hbm)
        plsc.subcore_barrier()

        @pl.when(sid < NS - 1)
        def _():
            pltpu.sync_copy(acc.at[sl], o_hbm.at[cid, sl])

        @pl.when(sid == NS - 1)
        def _():
            pltpu.sync_copy(acc.at[sl_last], o_hbm.at[cid, sl_last])

    return sk(vals, idx, zeros)


def _enc_body(x_ref, aemb_ref, w1_ref, xh_ref, xw1_ref):
    rows = x_ref.shape[0]
    acc = jnp.zeros((rows, H), jnp.float32)
    for f in range(AF):
        oh = (x_ref[:, f:f + 1] ==
              lax.broadcasted_iota(jnp.int32, (rows, AV), 1)).astype(jnp.float32)
        acc = acc + lax.dot_general(oh, aemb_ref[f], (((1,), (0,)), ((), ())),
                                    precision=_HIGH)
    xh_ref[...] = acc
    w1a = w1_ref[:, :H]
    xw1_ref[...] = lax.dot_general(acc, w1a, (((1,), (1,)), ((), ())),
                                   precision=_HIGH)


def _t512_body(bemb_ref, w1_ref, t_ref):
    w1b = w1_ref[:, H:]
    t0 = lax.dot_general(bemb_ref[0], w1b, (((1,), (1,)), ((), ())), precision=_HIGH)
    t1 = lax.dot_general(bemb_ref[1], w1b, (((1,), (1,)), ((), ())), precision=_HIGH)
    t2 = lax.dot_general(bemb_ref[2], w1b, (((1,), (1,)), ((), ())), precision=_HIGH)
    t = (t0[:, None, None, :] + t1[None, :, None, :] + t2[None, None, :, :])
    t_ref[...] = t.reshape(T, H)


def _h0_body(gx_ref, c_ref, t_ref, r_ref, w2_ref, h0_ref, hw0_ref):
    rows = gx_ref.shape[0]
    oh = (c_ref[...] == lax.broadcasted_iota(jnp.int32, (rows, T), 1)
          ).astype(jnp.bfloat16)
    gt = lax.dot_general(oh, t_ref[...].astype(jnp.bfloat16),
                         (((1,), (0,)), ((), ())),
                         preferred_element_type=jnp.float32)
    h0 = jax.nn.relu(gx_ref[...] + gt) + r_ref[...]
    h0_ref[...] = h0.astype(jnp.bfloat16)
    hw0_ref[...] = lax.dot_general(h0, w2_ref[...], (((1,), (1,)), ((), ())),
                                   precision=_HIGH)


def _comb_body(p_ref, o_ref):
    o_ref[...] = p_ref[0] + p_ref[1]


def _layer_body(h0_ref, gm_ref, gr_ref, w2_ref, hw_ref):
    h = jax.nn.relu(h0_ref[...].astype(jnp.float32) + gm_ref[...] - gr_ref[...])
    hw_ref[...] = lax.dot_general(h, w2_ref[...], (((1,), (1,)), ((), ())),
                                  precision=_HIGH)


def _last_body(h0_ref, gm_ref, gr_ref, h_ref):
    h_ref[...] = jax.nn.relu(h0_ref[...].astype(jnp.float32)
                             + gm_ref[...] - gr_ref[...])


def _post_body(xh_ref, p_ref, b_ref, w3_ref, b3_ref, s_ref, c_ref):
    i = pl.program_id(0)
    rows = xh_ref.shape[0]
    v = p_ref[0] + p_ref[1]
    na = jax.nn.relu(
        lax.dot_general(xh_ref[...], w3_ref[:, :H], (((1,), (1,)), ((), ())),
                        precision=_HIGH)
        + lax.dot_general(v, w3_ref[:, H:], (((1,), (1,)), ((), ())),
                          precision=_HIGH)
        + b3_ref[...])
    oht = (b_ref[0] == lax.broadcasted_iota(jnp.int32, (G, rows), 0)
           ).astype(jnp.float32)
    s_new = lax.dot_general(oht, na, (((1,), (0,)), ((), ())), precision=_HIGH)
    c_new = lax.dot_general(oht, jnp.ones((rows, H), jnp.float32),
                            (((1,), (0,)), ((), ())), precision=_HIGH)

    @pl.when(i == 0)
    def _():
        s_ref[...] = jnp.zeros_like(s_ref)
        c_ref[...] = jnp.zeros_like(c_ref)

    s_ref[...] += s_new
    c_ref[...] += c_new


_NB_NODE = 10
_NR = N // _NB_NODE
_NB_EDGE = 160
_ER = E // _NB_EDGE


def _f32_spec(rows, cols):
    return pl.BlockSpec((rows, cols), lambda i: (i, 0))


def _const_spec(shape):
    nd = len(shape)
    return pl.BlockSpec(shape, lambda i: (0,) * nd)


def kernel(x, edge_index, revedge_index, edge_attr, num_nodes, batch,
           atom_emb, bond_emb, W1, W2, W3, b3):
    f32 = jnp.float32
    src = edge_index[0].astype(jnp.int32)
    dst = edge_index[1].astype(jnp.int32)
    rev = revedge_index.astype(jnp.int32)
    ea = edge_attr.astype(jnp.int32)
    cidx = ea[:, 0] * (BV * BV) + ea[:, 1] * BV + ea[:, 2]

    src2 = src.reshape(NBLK, GC)
    dst2 = dst.reshape(NBLK, GC)
    rev2 = rev.reshape(NBLK, GC)
    cidx2 = cidx.reshape(E, 1)

    resid = (jnp.asarray(num_nodes) - N).astype(f32)
    rvec = jnp.full((1, H), resid, f32)
    zeros_n = jnp.zeros((N, H), f32)
    batch3 = batch.astype(jnp.int32).reshape(_NB_NODE, 1, _NR)

    x_h, xw1 = pl.pallas_call(
        _enc_body,
        grid=(_NB_NODE,),
        in_specs=[pl.BlockSpec((_NR, AF), lambda i: (i, 0)),
                  _const_spec((AF, AV, H)),
                  _const_spec((H, 2 * H))],
        out_specs=[_f32_spec(_NR, H), _f32_spec(_NR, H)],
        out_shape=[jax.ShapeDtypeStruct((N, H), f32),
                   jax.ShapeDtypeStruct((N, H), f32)],
    )(x, atom_emb, W1)

    t512 = pl.pallas_call(
        _t512_body,
        in_specs=[pl.BlockSpec((BF, BV, H), lambda: (0, 0, 0)),
                  pl.BlockSpec((H, 2 * H), lambda: (0, 0))],
        out_specs=pl.BlockSpec((T, H), lambda: (0, 0)),
        out_shape=jax.ShapeDtypeStruct((T, H), f32),
    )(bond_emb, W1)

    gx = _sc_gather1(xw1, src2)
    h0, hw = pl.pallas_call(
        _h0_body,
        grid=(_NB_EDGE,),
        in_specs=[_f32_spec(_ER, H),
                  pl.BlockSpec((_ER, 1), lambda i: (i, 0)),
                  _const_spec((T, H)),
                  _const_spec((1, H)), _const_spec((H, H))],
        out_specs=[_f32_spec(_ER, H), _f32_spec(_ER, H)],
        out_shape=[jax.ShapeDtypeStruct((E, H), jnp.bfloat16),
                   jax.ShapeDtypeStruct((E, H), f32)],
    )(gx, cidx2, t512, rvec, W2)

    for layer in range(2):
        parts = _sc_scatter_add(hw, dst2, zeros_n)
        mw = pl.pallas_call(
            _comb_body,
            grid=(_NB_NODE,),
            in_specs=[pl.BlockSpec((NC, _NR, H), lambda i: (0, i, 0))],
            out_specs=_f32_spec(_NR, H),
            out_shape=jax.ShapeDtypeStruct((N, H), f32),
        )(parts)
        gm, gr = _sc_gather2(mw, src2, hw, rev2)
        if layer == 0:
            hw = pl.pallas_call(
                _layer_body,
                grid=(_NB_EDGE,),
                in_specs=[_f32_spec(_ER, H), _f32_spec(_ER, H),
                          _f32_spec(_ER, H), _const_spec((H, H))],
                out_specs=_f32_spec(_ER, H),
                out_shape=jax.ShapeDtypeStruct((E, H), f32),
            )(h0, gm, gr, W2)
        else:
            h_last = pl.pallas_call(
                _last_body,
                grid=(_NB_EDGE,),
                in_specs=[_f32_spec(_ER, H), _f32_spec(_ER, H),
                          _f32_spec(_ER, H)],
                out_specs=_f32_spec(_ER, H),
                out_shape=jax.ShapeDtypeStruct((E, H), f32),
            )(h0, gm, gr)

    parts = _sc_scatter_add(h_last, dst2, zeros_n)
    sums, counts = pl.pallas_call(
        _post_body,
        grid=(_NB_NODE,),
        in_specs=[_f32_spec(_NR, H),
                  pl.BlockSpec((NC, _NR, H), lambda i: (0, i, 0)),
                  pl.BlockSpec((1, 1, _NR), lambda i: (i, 0, 0)),
                  _const_spec((H, 2 * H)),
                  _const_spec((1, H))],
        out_specs=[_const_spec((G, H)), _const_spec((G, H))],
        out_shape=[jax.ShapeDtypeStruct((G, H), f32),
                   jax.ShapeDtypeStruct((G, H), f32)],
    )(x_h, parts, batch3, W3, b3.reshape(1, H))

    return sums / jnp.clip(counts, 1.0)


# speedup vs baseline: 1.0836x; 1.0836x over previous
import functools

import jax
import jax.numpy as jnp
from jax import lax
from jax.experimental import pallas as pl
from jax.experimental.pallas import tpu as pltpu
from jax.experimental.pallas import tpu_sc as plsc

H = 128
N = 10000
E = 320000
G = 64
AF, AV = 9, 64
BF, BV = 3, 8
T = BV ** BF

GC = 128
NBLK = E // GC
NC, NS = 2, 16
ROWS_PER_SUB = 624

_HIGH = jax.lax.Precision.HIGHEST

_mesh = plsc.VectorSubcoreMesh(core_axis_name="c", subcore_axis_name="s")


def _sc_gather1(tab, idx):
    @functools.partial(
        pl.kernel,
        out_type=jax.ShapeDtypeStruct((E, H), tab.dtype),
        mesh=_mesh,
    )
    def g1(t_hbm, i_hbm, o_hbm):
        def body(i_v, o_v):
            pltpu.sync_copy(t_hbm.at[i_v.at[0]], o_v)

        pltpu.emit_pipeline(
            body,
            grid=(NBLK,),
            in_specs=[pl.BlockSpec((1, GC), lambda i: (i, 0))],
            out_specs=[pl.BlockSpec((GC, H), lambda i: (i, 0))],
            core_axis_name=("c", "s"),
            dimension_semantics=(pltpu.PARALLEL,),
        )(i_hbm, o_hbm)

    return g1(tab, idx)


def _sc_gather2(tab_a, idx_a, tab_b, idx_b):
    @functools.partial(
        pl.kernel,
        out_type=(jax.ShapeDtypeStruct((E, H), tab_a.dtype),
                  jax.ShapeDtypeStruct((E, H), tab_b.dtype)),
        mesh=_mesh,
    )
    def g2(ta_hbm, ia_hbm, tb_hbm, ib_hbm, oa_hbm, ob_hbm):
        def body(ia_v, ib_v, oa_v, ob_v):
            pltpu.sync_copy(ta_hbm.at[ia_v.at[0]], oa_v)
            pltpu.sync_copy(tb_hbm.at[ib_v.at[0]], ob_v)

        pltpu.emit_pipeline(
            body,
            grid=(NBLK,),
            in_specs=[
                pl.BlockSpec((1, GC), lambda i: (i, 0)),
                pl.BlockSpec((1, GC), lambda i: (i, 0)),
            ],
            out_specs=[
                pl.BlockSpec((GC, H), lambda i: (i, 0)),
                pl.BlockSpec((GC, H), lambda i: (i, 0)),
            ],
            core_axis_name=("c", "s"),
            dimension_semantics=(pltpu.PARALLEL,),
        )(ia_hbm, ib_hbm, oa_hbm, ob_hbm)

    return g2(tab_a, idx_a, tab_b, idx_b)


def _sc_scatter_add(vals, idx, zeros):
    @functools.partial(
        pl.kernel,
        out_type=jax.ShapeDtypeStruct((NC, N, H), jnp.float32),
        mesh=_mesh,
        scratch_types=[pltpu.VMEM_SHARED((N, H), jnp.float32)],
    )
    def sk(v_hbm, i_hbm, z_hbm, o_hbm, acc):
        cid = lax.axis_index("c")
        sid = lax.axis_index("s")
        sl = pl.ds(sid * ROWS_PER_SUB, ROWS_PER_SUB)
        sl_last = pl.ds((NS - 1) * ROWS_PER_SUB, N - (NS - 1) * ROWS_PER_SUB)

        @pl.when(sid < NS - 1)
        def _():
            pltpu.sync_copy(z_hbm.at[sl], acc.at[sl])

        @pl.when(sid == NS - 1)
        def _():
            pltpu.sync_copy(z_hbm.at[sl_last], acc.at[sl_last])

        plsc.subcore_barrier()

        def body(v_v, i_v):
            pltpu.sync_copy(v_v, acc.at[i_v.at[0]], add=True)

        pltpu.emit_pipeline(
            body,
            grid=(NBLK,),
            in_specs=[
                pl.BlockSpec((GC, H), lambda i: (i, 0)),
                pl.BlockSpec((1, GC), lambda i: (i, 0)),
            ],
            out_specs=[],
            core_axis_name=("c", "s"),
            dimension_semantics=(pltpu.PARALLEL,),
        )(v_hbm, i_hbm)
        plsc.subcore_barrier()

        @pl.when(sid < NS - 1)
        def _():
            pltpu.sync_copy(acc.at[sl], o_hbm.at[cid, sl])

        @pl.when(sid == NS - 1)
        def _():
            pltpu.sync_copy(acc.at[sl_last], o_hbm.at[cid, sl_last])

    return sk(vals, idx, zeros)


def _enc_body(x_ref, aemb_ref, w1_ref, xh_ref, xw1_ref):
    rows = x_ref.shape[0]
    acc = jnp.zeros((rows, H), jnp.float32)
    for f in range(AF):
        oh = (x_ref[:, f:f + 1] ==
              lax.broadcasted_iota(jnp.int32, (rows, AV), 1)).astype(jnp.float32)
        acc = acc + lax.dot_general(oh, aemb_ref[f], (((1,), (0,)), ((), ())),
                                    precision=_HIGH)
    xh_ref[...] = acc
    w1a = w1_ref[:, :H]
    xw1_ref[...] = lax.dot_general(acc, w1a, (((1,), (1,)), ((), ())),
                                   precision=_HIGH)


def _t512_body(bemb_ref, w1_ref, t_ref):
    w1b = w1_ref[:, H:]
    t0 = lax.dot_general(bemb_ref[0], w1b, (((1,), (1,)), ((), ())), precision=_HIGH)
    t1 = lax.dot_general(bemb_ref[1], w1b, (((1,), (1,)), ((), ())), precision=_HIGH)
    t2 = lax.dot_general(bemb_ref[2], w1b, (((1,), (1,)), ((), ())), precision=_HIGH)
    t = (t0[:, None, None, :] + t1[None, :, None, :] + t2[None, None, :, :])
    t_ref[...] = t.reshape(T, H)


def _h0_body(gx_ref, gt_ref, r_ref, w2_ref, h0_ref, hw0_ref):
    h0 = jax.nn.relu(gx_ref[...] + gt_ref[...]) + r_ref[...]
    h0_ref[...] = h0.astype(jnp.bfloat16)
    hw0_ref[...] = lax.dot_general(h0, w2_ref[...], (((1,), (1,)), ((), ())),
                                   precision=_HIGH)


def _comb_body(p_ref, o_ref):
    o_ref[...] = p_ref[0] + p_ref[1]


def _layer_body(h0_ref, gm_ref, gr_ref, w2_ref, hw_ref):
    h = jax.nn.relu(h0_ref[...].astype(jnp.float32) + gm_ref[...] - gr_ref[...])
    hw_ref[...] = lax.dot_general(h, w2_ref[...], (((1,), (1,)), ((), ())),
                                  precision=_HIGH)


def _last_body(h0_ref, gm_ref, gr_ref, h_ref):
    h_ref[...] = jax.nn.relu(h0_ref[...].astype(jnp.float32)
                             + gm_ref[...] - gr_ref[...])


def _post_body(xh_ref, p_ref, b_ref, w3_ref, b3_ref, s_ref, c_ref):
    i = pl.program_id(0)
    rows = xh_ref.shape[0]
    v = p_ref[0] + p_ref[1]
    na = jax.nn.relu(
        lax.dot_general(xh_ref[...], w3_ref[:, :H], (((1,), (1,)), ((), ())),
                        precision=_HIGH)
        + lax.dot_general(v, w3_ref[:, H:], (((1,), (1,)), ((), ())),
                          precision=_HIGH)
        + b3_ref[...])
    oht = (b_ref[0] == lax.broadcasted_iota(jnp.int32, (G, rows), 0)
           ).astype(jnp.float32)
    s_new = lax.dot_general(oht, na, (((1,), (0,)), ((), ())), precision=_HIGH)
    c_new = lax.dot_general(oht, jnp.ones((rows, H), jnp.float32),
                            (((1,), (0,)), ((), ())), precision=_HIGH)

    @pl.when(i == 0)
    def _():
        s_ref[...] = jnp.zeros_like(s_ref)
        c_ref[...] = jnp.zeros_like(c_ref)

    s_ref[...] += s_new
    c_ref[...] += c_new


_NB_NODE = 10
_NR = N // _NB_NODE
_NB_EDGE = 160
_ER = E // _NB_EDGE


def _f32_spec(rows, cols):
    return pl.BlockSpec((rows, cols), lambda i: (i, 0))


def _const_spec(shape):
    nd = len(shape)
    return pl.BlockSpec(shape, lambda i: (0,) * nd)


def kernel(x, edge_index, revedge_index, edge_attr, num_nodes, batch,
           atom_emb, bond_emb, W1, W2, W3, b3):
    f32 = jnp.float32
    src = edge_index[0].astype(jnp.int32)
    dst = edge_index[1].astype(jnp.int32)
    rev = revedge_index.astype(jnp.int32)
    ea = edge_attr.astype(jnp.int32)
    cidx = ea[:, 0] * (BV * BV) + ea[:, 1] * BV + ea[:, 2]

    src2 = src.reshape(NBLK, GC)
    dst2 = dst.reshape(NBLK, GC)
    rev2 = rev.reshape(NBLK, GC)
    cidx2 = cidx.reshape(NBLK, GC)

    resid = (jnp.asarray(num_nodes) - N).astype(f32)
    rvec = jnp.full((1, H), resid, f32)
    zeros_n = jnp.zeros((N, H), f32)
    batch3 = batch.astype(jnp.int32).reshape(_NB_NODE, 1, _NR)

    x_h, xw1 = pl.pallas_call(
        _enc_body,
        grid=(_NB_NODE,),
        in_specs=[pl.BlockSpec((_NR, AF), lambda i: (i, 0)),
                  _const_spec((AF, AV, H)),
                  _const_spec((H, 2 * H))],
        out_specs=[_f32_spec(_NR, H), _f32_spec(_NR, H)],
        out_shape=[jax.ShapeDtypeStruct((N, H), f32),
                   jax.ShapeDtypeStruct((N, H), f32)],
    )(x, atom_emb, W1)

    t512 = pl.pallas_call(
        _t512_body,
        in_specs=[pl.BlockSpec((BF, BV, H), lambda: (0, 0, 0)),
                  pl.BlockSpec((H, 2 * H), lambda: (0, 0))],
        out_specs=pl.BlockSpec((T, H), lambda: (0, 0)),
        out_shape=jax.ShapeDtypeStruct((T, H), f32),
    )(bond_emb, W1)

    gx, gt = _sc_gather2(xw1, src2, t512, cidx2)
    h0, hw = pl.pallas_call(
        _h0_body,
        grid=(_NB_EDGE,),
        in_specs=[_f32_spec(_ER, H), _f32_spec(_ER, H),
                  _const_spec((1, H)), _const_spec((H, H))],
        out_specs=[_f32_spec(_ER, H), _f32_spec(_ER, H)],
        out_shape=[jax.ShapeDtypeStruct((E, H), jnp.bfloat16),
                   jax.ShapeDtypeStruct((E, H), f32)],
    )(gx, gt, rvec, W2)

    for layer in range(2):
        parts = _sc_scatter_add(hw, dst2, zeros_n)
        mw = pl.pallas_call(
            _comb_body,
            grid=(_NB_NODE,),
            in_specs=[pl.BlockSpec((NC, _NR, H), lambda i: (0, i, 0))],
            out_specs=_f32_spec(_NR, H),
            out_shape=jax.ShapeDtypeStruct((N, H), f32),
        )(parts)
        gm, gr = _sc_gather2(mw, src2, hw, rev2)
        if layer == 0:
            hw = pl.pallas_call(
                _layer_body,
                grid=(_NB_EDGE,),
                in_specs=[_f32_spec(_ER, H), _f32_spec(_ER, H),
                          _f32_spec(_ER, H), _const_spec((H, H))],
                out_specs=_f32_spec(_ER, H),
                out_shape=jax.ShapeDtypeStruct((E, H), f32),
            )(h0, gm, gr, W2)
        else:
            h_last = pl.pallas_call(
                _last_body,
                grid=(_NB_EDGE,),
                in_specs=[_f32_spec(_ER, H), _f32_spec(_ER, H),
                          _f32_spec(_ER, H)],
                out_specs=_f32_spec(_ER, H),
                out_shape=jax.ShapeDtypeStruct((E, H), f32),
            )(h0, gm, gr)

    parts = _sc_scatter_add(h_last, dst2, zeros_n)
    sums, counts = pl.pallas_call(
        _post_body,
        grid=(_NB_NODE,),
        in_specs=[_f32_spec(_NR, H),
                  pl.BlockSpec((NC, _NR, H), lambda i: (0, i, 0)),
                  pl.BlockSpec((1, 1, _NR), lambda i: (i, 0, 0)),
                  _const_spec((H, 2 * H)),
                  _const_spec((1, H))],
        out_specs=[_const_spec((G, H)), _const_spec((G, H))],
        out_shape=[jax.ShapeDtypeStruct((G, H), f32),
                   jax.ShapeDtypeStruct((G, H), f32)],
    )(x_h, parts, batch3, W3, b3.reshape(1, H))

    return sums / jnp.clip(counts, 1.0)


# speedup vs baseline: 1.1382x; 1.0504x over previous
import functools

import jax
import jax.numpy as jnp
from jax import lax
from jax.experimental import pallas as pl
from jax.experimental.pallas import tpu as pltpu
from jax.experimental.pallas import tpu_sc as plsc

H = 128
N = 10000
E = 320000
G = 64
AF, AV = 9, 64
BF, BV = 3, 8
T = BV ** BF

GC = 128
NBLK = E // GC
NC, NS = 2, 16
ROWS_PER_SUB = 624

_HIGH = jax.lax.Precision.HIGHEST

_mesh = plsc.VectorSubcoreMesh(core_axis_name="c", subcore_axis_name="s")


def _sc_gather1(tab, idx):
    @functools.partial(
        pl.kernel,
        out_type=jax.ShapeDtypeStruct((E, H), tab.dtype),
        mesh=_mesh,
    )
    def g1(t_hbm, i_hbm, o_hbm):
        def body(i_v, o_v):
            pltpu.sync_copy(t_hbm.at[i_v.at[0]], o_v)

        pltpu.emit_pipeline(
            body,
            grid=(NBLK,),
            in_specs=[pl.BlockSpec((1, GC), lambda i: (i, 0))],
            out_specs=[pl.BlockSpec((GC, H), lambda i: (i, 0))],
            core_axis_name=("c", "s"),
            dimension_semantics=(pltpu.PARALLEL,),
        )(i_hbm, o_hbm)

    return g1(tab, idx)


def _sc_gather2(tab_a, idx_a, tab_b, idx_b):
    @functools.partial(
        pl.kernel,
        out_type=(jax.ShapeDtypeStruct((E, H), tab_a.dtype),
                  jax.ShapeDtypeStruct((E, H), tab_b.dtype)),
        mesh=_mesh,
        scratch_types=[pltpu.SemaphoreType.DMA],
    )
    def g2(ta_hbm, ia_hbm, tb_hbm, ib_hbm, oa_hbm, ob_hbm, sem):
        def body(ia_v, ib_v, oa_v, ob_v):
            d = pltpu.async_copy(ta_hbm.at[ia_v.at[0]], oa_v, sem)
            pltpu.sync_copy(tb_hbm.at[ib_v.at[0]], ob_v)
            d.wait()

        pltpu.emit_pipeline(
            body,
            grid=(NBLK,),
            in_specs=[
                pl.BlockSpec((1, GC), lambda i: (i, 0)),
                pl.BlockSpec((1, GC), lambda i: (i, 0)),
            ],
            out_specs=[
                pl.BlockSpec((GC, H), lambda i: (i, 0)),
                pl.BlockSpec((GC, H), lambda i: (i, 0)),
            ],
            core_axis_name=("c", "s"),
            dimension_semantics=(pltpu.PARALLEL,),
        )(ia_hbm, ib_hbm, oa_hbm, ob_hbm)

    return g2(tab_a, idx_a, tab_b, idx_b)


def _sc_scatter_add(vals, idx, zeros):
    @functools.partial(
        pl.kernel,
        out_type=jax.ShapeDtypeStruct((NC, N, H), jnp.float32),
        mesh=_mesh,
        scratch_types=[pltpu.VMEM_SHARED((N, H), jnp.float32)],
    )
    def sk(v_hbm, i_hbm, z_hbm, o_hbm, acc):
        cid = lax.axis_index("c")
        sid = lax.axis_index("s")
        sl = pl.ds(sid * ROWS_PER_SUB, ROWS_PER_SUB)
        sl_last = pl.ds((NS - 1) * ROWS_PER_SUB, N - (NS - 1) * ROWS_PER_SUB)

        @pl.when(sid < NS - 1)
        def _():
            pltpu.sync_copy(z_hbm.at[sl], acc.at[sl])

        @pl.when(sid == NS - 1)
        def _():
            pltpu.sync_copy(z_hbm.at[sl_last], acc.at[sl_last])

        plsc.subcore_barrier()

        def body(v_v, i_v):
            pltpu.sync_copy(v_v, acc.at[i_v.at[0]], add=True)

        pltpu.emit_pipeline(
            body,
            grid=(NBLK,),
            in_specs=[
                pl.BlockSpec((GC, H), lambda i: (i, 0)),
                pl.BlockSpec((1, GC), lambda i: (i, 0)),
            ],
            out_specs=[],
            core_axis_name=("c", "s"),
            dimension_semantics=(pltpu.PARALLEL,),
        )(v_hbm, i_hbm)
        plsc.subcore_barrier()

        @pl.when(sid < NS - 1)
        def _():
            pltpu.sync_copy(acc.at[sl], o_hbm.at[cid, sl])

        @pl.when(sid == NS - 1)
        def _():
            pltpu.sync_copy(acc.at[sl_last], o_hbm.at[cid, sl_last])

    return sk(vals, idx, zeros)


def _enc_body(x_ref, aemb_ref, w1_ref, xh_ref, xw1_ref):
    rows = x_ref.shape[0]
    acc = jnp.zeros((rows, H), jnp.float32)
    for f in range(AF):
        oh = (x_ref[:, f:f + 1] ==
              lax.broadcasted_iota(jnp.int32, (rows, AV), 1)).astype(jnp.float32)
        acc = acc + lax.dot_general(oh, aemb_ref[f], (((1,), (0,)), ((), ())),
                                    precision=_HIGH)
    xh_ref[...] = acc
    w1a = w1_ref[:, :H]
    xw1_ref[...] = lax.dot_general(acc, w1a, (((1,), (1,)), ((), ())),
                                   precision=_HIGH)


def _t512_body(bemb_ref, w1_ref, t_ref):
    w1b = w1_ref[:, H:]
    t0 = lax.dot_general(bemb_ref[0], w1b, (((1,), (1,)), ((), ())), precision=_HIGH)
    t1 = lax.dot_general(bemb_ref[1], w1b, (((1,), (1,)), ((), ())), precision=_HIGH)
    t2 = lax.dot_general(bemb_ref[2], w1b, (((1,), (1,)), ((), ())), precision=_HIGH)
    t = (t0[:, None, None, :] + t1[None, :, None, :] + t2[None, None, :, :])
    t_ref[...] = t.reshape(T, H)


def _h0_body(gx_ref, gt_ref, r_ref, w2_ref, h0_ref, hw0_ref):
    h0 = jax.nn.relu(gx_ref[...] + gt_ref[...]) + r_ref[...]
    h0_ref[...] = h0.astype(jnp.bfloat16)
    hw0_ref[...] = lax.dot_general(h0, w2_ref[...], (((1,), (1,)), ((), ())),
                                   precision=_HIGH)


def _comb_body(p_ref, o_ref):
    o_ref[...] = p_ref[0] + p_ref[1]


def _layer_body(h0_ref, gm_ref, gr_ref, w2_ref, hw_ref):
    h = jax.nn.relu(h0_ref[...].astype(jnp.float32) + gm_ref[...] - gr_ref[...])
    hw_ref[...] = lax.dot_general(h, w2_ref[...], (((1,), (1,)), ((), ())),
                                  precision=_HIGH)


def _last_body(h0_ref, gm_ref, gr_ref, h_ref):
    h_ref[...] = jax.nn.relu(h0_ref[...].astype(jnp.float32)
                             + gm_ref[...] - gr_ref[...])


def _post_body(xh_ref, p_ref, b_ref, w3_ref, b3_ref, s_ref, c_ref):
    i = pl.program_id(0)
    rows = xh_ref.shape[0]
    v = p_ref[0] + p_ref[1]
    na = jax.nn.relu(
        lax.dot_general(xh_ref[...], w3_ref[:, :H], (((1,), (1,)), ((), ())),
                        precision=_HIGH)
        + lax.dot_general(v, w3_ref[:, H:], (((1,), (1,)), ((), ())),
                          precision=_HIGH)
        + b3_ref[...])
    oht = (b_ref[0] == lax.broadcasted_iota(jnp.int32, (G, rows), 0)
           ).astype(jnp.float32)
    s_new = lax.dot_general(oht, na, (((1,), (0,)), ((), ())), precision=_HIGH)
    c_new = lax.dot_general(oht, jnp.ones((rows, H), jnp.float32),
                            (((1,), (0,)), ((), ())), precision=_HIGH)

    @pl.when(i == 0)
    def _():
        s_ref[...] = jnp.zeros_like(s_ref)
        c_ref[...] = jnp.zeros_like(c_ref)

    s_ref[...] += s_new
    c_ref[...] += c_new


_NB_NODE = 10
_NR = N // _NB_NODE
_NB_EDGE = 160
_ER = E // _NB_EDGE


def _f32_spec(rows, cols):
    return pl.BlockSpec((rows, cols), lambda i: (i, 0))


def _const_spec(shape):
    nd = len(shape)
    return pl.BlockSpec(shape, lambda i: (0,) * nd)


def kernel(x, edge_index, revedge_index, edge_attr, num_nodes, batch,
           atom_emb, bond_emb, W1, W2, W3, b3):
    f32 = jnp.float32
    src = edge_index[0].astype(jnp.int32)
    dst = edge_index[1].astype(jnp.int32)
    rev = revedge_index.astype(jnp.int32)
    ea = edge_attr.astype(jnp.int32)
    cidx = ea[:, 0] * (BV * BV) + ea[:, 1] * BV + ea[:, 2]

    src2 = src.reshape(NBLK, GC)
    dst2 = dst.reshape(NBLK, GC)
    rev2 = rev.reshape(NBLK, GC)
    cidx2 = cidx.reshape(NBLK, GC)

    resid = (jnp.asarray(num_nodes) - N).astype(f32)
    rvec = jnp.full((1, H), resid, f32)
    zeros_n = jnp.zeros((N, H), f32)
    batch3 = batch.astype(jnp.int32).reshape(_NB_NODE, 1, _NR)

    x_h, xw1 = pl.pallas_call(
        _enc_body,
        grid=(_NB_NODE,),
        in_specs=[pl.BlockSpec((_NR, AF), lambda i: (i, 0)),
                  _const_spec((AF, AV, H)),
                  _const_spec((H, 2 * H))],
        out_specs=[_f32_spec(_NR, H), _f32_spec(_NR, H)],
        out_shape=[jax.ShapeDtypeStruct((N, H), f32),
                   jax.ShapeDtypeStruct((N, H), f32)],
    )(x, atom_emb, W1)

    t512 = pl.pallas_call(
        _t512_body,
        in_specs=[pl.BlockSpec((BF, BV, H), lambda: (0, 0, 0)),
                  pl.BlockSpec((H, 2 * H), lambda: (0, 0))],
        out_specs=pl.BlockSpec((T, H), lambda: (0, 0)),
        out_shape=jax.ShapeDtypeStruct((T, H), f32),
    )(bond_emb, W1)

    gx, gt = _sc_gather2(xw1, src2, t512, cidx2)
    h0, hw = pl.pallas_call(
        _h0_body,
        grid=(_NB_EDGE,),
        in_specs=[_f32_spec(_ER, H), _f32_spec(_ER, H),
                  _const_spec((1, H)), _const_spec((H, H))],
        out_specs=[_f32_spec(_ER, H), _f32_spec(_ER, H)],
        out_shape=[jax.ShapeDtypeStruct((E, H), jnp.bfloat16),
                   jax.ShapeDtypeStruct((E, H), f32)],
    )(gx, gt, rvec, W2)

    for layer in range(2):
        parts = _sc_scatter_add(hw, dst2, zeros_n)
        mw = pl.pallas_call(
            _comb_body,
            grid=(_NB_NODE,),
            in_specs=[pl.BlockSpec((NC, _NR, H), lambda i: (0, i, 0))],
            out_specs=_f32_spec(_NR, H),
            out_shape=jax.ShapeDtypeStruct((N, H), f32),
        )(parts)
        gm, gr = _sc_gather2(mw, src2, hw, rev2)
        if layer == 0:
            hw = pl.pallas_call(
                _layer_body,
                grid=(_NB_EDGE,),
                in_specs=[_f32_spec(_ER, H), _f32_spec(_ER, H),
                          _f32_spec(_ER, H), _const_spec((H, H))],
                out_specs=_f32_spec(_ER, H),
                out_shape=jax.ShapeDtypeStruct((E, H), f32),
            )(h0, gm, gr, W2)
        else:
            h_last = pl.pallas_call(
                _last_body,
                grid=(_NB_EDGE,),
                in_specs=[_f32_spec(_ER, H), _f32_spec(_ER, H),
                          _f32_spec(_ER, H)],
                out_specs=_f32_spec(_ER, H),
                out_shape=jax.ShapeDtypeStruct((E, H), f32),
            )(h0, gm, gr)

    parts = _sc_scatter_add(h_last, dst2, zeros_n)
    sums, counts = pl.pallas_call(
        _post_body,
        grid=(_NB_NODE,),
        in_specs=[_f32_spec(_NR, H),
                  pl.BlockSpec((NC, _NR, H), lambda i: (0, i, 0)),
                  pl.BlockSpec((1, 1, _NR), lambda i: (i, 0, 0)),
                  _const_spec((H, 2 * H)),
                  _const_spec((1, H))],
        out_specs=[_const_spec((G, H)), _const_spec((G, H))],
        out_shape=[jax.ShapeDtypeStruct((G, H), f32),
                   jax.ShapeDtypeStruct((G, H), f32)],
    )(x_h, parts, batch3, W3, b3.reshape(1, H))

    return sums / jnp.clip(counts, 1.0)


# speedup vs baseline: 1.1892x; 1.0448x over previous
import functools

import jax
import jax.numpy as jnp
from jax import lax
from jax.experimental import pallas as pl
from jax.experimental.pallas import tpu as pltpu
from jax.experimental.pallas import tpu_sc as plsc

H = 128
N = 10000
E = 320000
G = 64
AF, AV = 9, 64
BF, BV = 3, 8
T = BV ** BF

GC = 128
NBLK = E // GC
NC, NS = 2, 16
ROWS_PER_SUB = 624

_HIGH = jax.lax.Precision.HIGHEST

_mesh = plsc.VectorSubcoreMesh(core_axis_name="c", subcore_axis_name="s")


def _sc_gather1(tab, idx):
    @functools.partial(
        pl.kernel,
        out_type=jax.ShapeDtypeStruct((E, H), tab.dtype),
        mesh=_mesh,
    )
    def g1(t_hbm, i_hbm, o_hbm):
        def body(i_v, o_v):
            pltpu.sync_copy(t_hbm.at[i_v.at[0]], o_v)

        pltpu.emit_pipeline(
            body,
            grid=(NBLK,),
            in_specs=[pl.BlockSpec((1, GC), lambda i: (i, 0))],
            out_specs=[pl.BlockSpec((GC, H), lambda i: (i, 0))],
            core_axis_name=("c", "s"),
            dimension_semantics=(pltpu.PARALLEL,),
        )(i_hbm, o_hbm)

    return g1(tab, idx)


def _sc_gather2(tab_a, idx_a, tab_b, idx_b):
    @functools.partial(
        pl.kernel,
        out_type=(jax.ShapeDtypeStruct((E, H), tab_a.dtype),
                  jax.ShapeDtypeStruct((E, H), tab_b.dtype)),
        mesh=_mesh,
        scratch_types=[pltpu.SemaphoreType.DMA],
    )
    def g2(ta_hbm, ia_hbm, tb_hbm, ib_hbm, oa_hbm, ob_hbm, sem):
        def body(ia_v, ib_v, oa_v, ob_v):
            d = pltpu.async_copy(ta_hbm.at[ia_v.at[0]], oa_v, sem)
            pltpu.sync_copy(tb_hbm.at[ib_v.at[0]], ob_v)
            d.wait()

        pltpu.emit_pipeline(
            body,
            grid=(NBLK,),
            in_specs=[
                pl.BlockSpec((1, GC), lambda i: (i, 0)),
                pl.BlockSpec((1, GC), lambda i: (i, 0)),
            ],
            out_specs=[
                pl.BlockSpec((GC, H), lambda i: (i, 0)),
                pl.BlockSpec((GC, H), lambda i: (i, 0)),
            ],
            core_axis_name=("c", "s"),
            dimension_semantics=(pltpu.PARALLEL,),
        )(ia_hbm, ib_hbm, oa_hbm, ob_hbm)

    return g2(tab_a, idx_a, tab_b, idx_b)


def _sc_scatter_add(vals, idx, zeros):
    @functools.partial(
        pl.kernel,
        out_type=jax.ShapeDtypeStruct((NC, N, H), jnp.float32),
        mesh=_mesh,
        scratch_types=[pltpu.VMEM_SHARED((N, H), jnp.float32)],
    )
    def sk(v_hbm, i_hbm, z_hbm, o_hbm, acc):
        cid = lax.axis_index("c")
        sid = lax.axis_index("s")
        sl = pl.ds(sid * ROWS_PER_SUB, ROWS_PER_SUB)
        sl_last = pl.ds((NS - 1) * ROWS_PER_SUB, N - (NS - 1) * ROWS_PER_SUB)

        @pl.when(sid < NS - 1)
        def _():
            pltpu.sync_copy(z_hbm.at[sl], acc.at[sl])

        @pl.when(sid == NS - 1)
        def _():
            pltpu.sync_copy(z_hbm.at[sl_last], acc.at[sl_last])

        plsc.subcore_barrier()

        def body(v_v, i_v):
            pltpu.sync_copy(v_v, acc.at[i_v.at[0]], add=True)

        pltpu.emit_pipeline(
            body,
            grid=(NBLK,),
            in_specs=[
                pl.BlockSpec((GC, H), lambda i: (i, 0)),
                pl.BlockSpec((1, GC), lambda i: (i, 0)),
            ],
            out_specs=[],
            core_axis_name=("c", "s"),
            dimension_semantics=(pltpu.PARALLEL,),
        )(v_hbm, i_hbm)
        plsc.subcore_barrier()

        @pl.when(sid < NS - 1)
        def _():
            pltpu.sync_copy(acc.at[sl], o_hbm.at[cid, sl])

        @pl.when(sid == NS - 1)
        def _():
            pltpu.sync_copy(acc.at[sl_last], o_hbm.at[cid, sl_last])

    return sk(vals, idx, zeros)


def _enc_body(x_ref, aemb_ref, w1_ref, xh_ref, xw1_ref):
    rows = x_ref.shape[0]
    acc = jnp.zeros((rows, H), jnp.float32)
    for f in range(AF):
        oh = (x_ref[:, f:f + 1] ==
              lax.broadcasted_iota(jnp.int32, (rows, AV), 1)).astype(jnp.float32)
        acc = acc + lax.dot_general(oh, aemb_ref[f], (((1,), (0,)), ((), ())),
                                    precision=_HIGH)
    xh_ref[...] = acc
    w1a = w1_ref[:, :H]
    xw1_ref[...] = lax.dot_general(acc, w1a, (((1,), (1,)), ((), ())),
                                   precision=_HIGH)


def _t512_body(bemb_ref, w1_ref, t_ref):
    w1b = w1_ref[:, H:]
    t0 = lax.dot_general(bemb_ref[0], w1b, (((1,), (1,)), ((), ())), precision=_HIGH)
    t1 = lax.dot_general(bemb_ref[1], w1b, (((1,), (1,)), ((), ())), precision=_HIGH)
    t2 = lax.dot_general(bemb_ref[2], w1b, (((1,), (1,)), ((), ())), precision=_HIGH)
    t = (t0[:, None, None, :] + t1[None, :, None, :] + t2[None, None, :, :])
    t_ref[...] = t.reshape(T, H)


def _h0_body(gx_ref, gt_ref, r_ref, w2_ref, h0_ref, hw0_ref):
    h0 = jax.nn.relu(gx_ref[...] + gt_ref[...]) + r_ref[...]
    h0_ref[...] = h0.astype(jnp.bfloat16)
    hw0_ref[...] = lax.dot_general(h0.astype(jnp.bfloat16),
                                   w2_ref[...].astype(jnp.bfloat16),
                                   (((1,), (1,)), ((), ())),
                                   preferred_element_type=jnp.float32)


def _comb_body(p_ref, o_ref):
    o_ref[...] = p_ref[0] + p_ref[1]


def _layer_body(h0_ref, gm_ref, gr_ref, w2_ref, hw_ref):
    h = jax.nn.relu(h0_ref[...].astype(jnp.float32) + gm_ref[...] - gr_ref[...])
    hw_ref[...] = lax.dot_general(h.astype(jnp.bfloat16),
                                  w2_ref[...].astype(jnp.bfloat16),
                                  (((1,), (1,)), ((), ())),
                                  preferred_element_type=jnp.float32)


def _last_body(h0_ref, gm_ref, gr_ref, h_ref):
    h_ref[...] = jax.nn.relu(h0_ref[...].astype(jnp.float32)
                             + gm_ref[...] - gr_ref[...])


def _post_body(xh_ref, p_ref, b_ref, w3_ref, b3_ref, s_ref, c_ref):
    i = pl.program_id(0)
    rows = xh_ref.shape[0]
    v = p_ref[0] + p_ref[1]
    na = jax.nn.relu(
        lax.dot_general(xh_ref[...], w3_ref[:, :H], (((1,), (1,)), ((), ())),
                        precision=_HIGH)
        + lax.dot_general(v, w3_ref[:, H:], (((1,), (1,)), ((), ())),
                          precision=_HIGH)
        + b3_ref[...])
    oht = (b_ref[0] == lax.broadcasted_iota(jnp.int32, (G, rows), 0)
           ).astype(jnp.float32)
    s_new = lax.dot_general(oht, na, (((1,), (0,)), ((), ())), precision=_HIGH)
    c_new = lax.dot_general(oht, jnp.ones((rows, H), jnp.float32),
                            (((1,), (0,)), ((), ())), precision=_HIGH)

    @pl.when(i == 0)
    def _():
        s_ref[...] = jnp.zeros_like(s_ref)
        c_ref[...] = jnp.zeros_like(c_ref)

    s_ref[...] += s_new
    c_ref[...] += c_new


_NB_NODE = 10
_NR = N // _NB_NODE
_NB_EDGE = 160
_ER = E // _NB_EDGE


def _f32_spec(rows, cols):
    return pl.BlockSpec((rows, cols), lambda i: (i, 0))


def _const_spec(shape):
    nd = len(shape)
    return pl.BlockSpec(shape, lambda i: (0,) * nd)


def kernel(x, edge_index, revedge_index, edge_attr, num_nodes, batch,
           atom_emb, bond_emb, W1, W2, W3, b3):
    f32 = jnp.float32
    src = edge_index[0].astype(jnp.int32)
    dst = edge_index[1].astype(jnp.int32)
    rev = revedge_index.astype(jnp.int32)
    ea = edge_attr.astype(jnp.int32)
    cidx = ea[:, 0] * (BV * BV) + ea[:, 1] * BV + ea[:, 2]

    src2 = src.reshape(NBLK, GC)
    dst2 = dst.reshape(NBLK, GC)
    rev2 = rev.reshape(NBLK, GC)
    cidx2 = cidx.reshape(NBLK, GC)

    resid = (jnp.asarray(num_nodes) - N).astype(f32)
    rvec = jnp.full((1, H), resid, f32)
    zeros_n = jnp.zeros((N, H), f32)
    batch3 = batch.astype(jnp.int32).reshape(_NB_NODE, 1, _NR)

    x_h, xw1 = pl.pallas_call(
        _enc_body,
        grid=(_NB_NODE,),
        in_specs=[pl.BlockSpec((_NR, AF), lambda i: (i, 0)),
                  _const_spec((AF, AV, H)),
                  _const_spec((H, 2 * H))],
        out_specs=[_f32_spec(_NR, H), _f32_spec(_NR, H)],
        out_shape=[jax.ShapeDtypeStruct((N, H), f32),
                   jax.ShapeDtypeStruct((N, H), f32)],
    )(x, atom_emb, W1)

    t512 = pl.pallas_call(
        _t512_body,
        in_specs=[pl.BlockSpec((BF, BV, H), lambda: (0, 0, 0)),
                  pl.BlockSpec((H, 2 * H), lambda: (0, 0))],
        out_specs=pl.BlockSpec((T, H), lambda: (0, 0)),
        out_shape=jax.ShapeDtypeStruct((T, H), f32),
    )(bond_emb, W1)

    gx, gt = _sc_gather2(xw1, src2, t512, cidx2)
    h0, hw = pl.pallas_call(
        _h0_body,
        grid=(_NB_EDGE,),
        in_specs=[_f32_spec(_ER, H), _f32_spec(_ER, H),
                  _const_spec((1, H)), _const_spec((H, H))],
        out_specs=[_f32_spec(_ER, H), _f32_spec(_ER, H)],
        out_shape=[jax.ShapeDtypeStruct((E, H), jnp.bfloat16),
                   jax.ShapeDtypeStruct((E, H), f32)],
    )(gx, gt, rvec, W2)

    for layer in range(2):
        parts = _sc_scatter_add(hw, dst2, zeros_n)
        mw = pl.pallas_call(
            _comb_body,
            grid=(_NB_NODE,),
            in_specs=[pl.BlockSpec((NC, _NR, H), lambda i: (0, i, 0))],
            out_specs=_f32_spec(_NR, H),
            out_shape=jax.ShapeDtypeStruct((N, H), f32),
        )(parts)
        gm, gr = _sc_gather2(mw, src2, hw, rev2)
        if layer == 0:
            hw = pl.pallas_call(
                _layer_body,
                grid=(_NB_EDGE,),
                in_specs=[_f32_spec(_ER, H), _f32_spec(_ER, H),
                          _f32_spec(_ER, H), _const_spec((H, H))],
                out_specs=_f32_spec(_ER, H),
                out_shape=jax.ShapeDtypeStruct((E, H), f32),
            )(h0, gm, gr, W2)
        else:
            h_last = pl.pallas_call(
                _last_body,
                grid=(_NB_EDGE,),
                in_specs=[_f32_spec(_ER, H), _f32_spec(_ER, H),
                          _f32_spec(_ER, H)],
                out_specs=_f32_spec(_ER, H),
                out_shape=jax.ShapeDtypeStruct((E, H), f32),
            )(h0, gm, gr)

    parts = _sc_scatter_add(h_last, dst2, zeros_n)
    sums, counts = pl.pallas_call(
        _post_body,
        grid=(_NB_NODE,),
        in_specs=[_f32_spec(_NR, H),
                  pl.BlockSpec((NC, _NR, H), lambda i: (0, i, 0)),
                  pl.BlockSpec((1, 1, _NR), lambda i: (i, 0, 0)),
                  _const_spec((H, 2 * H)),
                  _const_spec((1, H))],
        out_specs=[_const_spec((G, H)), _const_spec((G, H))],
        out_shape=[jax.ShapeDtypeStruct((G, H), f32),
                   jax.ShapeDtypeStruct((G, H), f32)],
    )(x_h, parts, batch3, W3, b3.reshape(1, H))

    return sums / jnp.clip(counts, 1.0)
